# baseline (speedup 1.0000x reference)
import functools

import jax
import jax.numpy as jnp
from jax import lax
from jax.experimental import pallas as pl
from jax.experimental.pallas import tpu as pltpu
from jax.experimental.pallas import tpu_sc as plsc

B = 4096
D_IN = 128
OUT_DIM = 100000
TOPK = 128
N_PAD = 100352
K_PAD = 256
BM = 256
BN = 512
CS = 32
NCHUNK = N_PAD // CS
NW = 32
ROWS_PER_W = B // NW
CAP = 256


def _mm_kernel(x_ref, w_ref, o_ref, mt_ref):
    acc = jax.lax.dot_general(
        x_ref[...], w_ref[...], (((1,), (1,)), ((), ())),
        preferred_element_type=jnp.float32)
    for c in range(BN // 128):
        o_ref[c] = acc[:, c * 128:(c + 1) * 128]
    m4 = jnp.max(acc.reshape(BM, BN // CS, CS), axis=2)
    mt_ref[...] = m4.T


def _scores_and_maxima(xa, Wa):
    grid = (N_PAD // BN, B // BM)
    return pl.pallas_call(
        _mm_kernel,
        grid=grid,
        in_specs=[
            pl.BlockSpec((BM, K_PAD), lambda j, i: (i, 0)),
            pl.BlockSpec((BN, K_PAD), lambda j, i: (j, 0)),
        ],
        out_specs=[
            pl.BlockSpec((BN // 128, BM, 128), lambda j, i: (j, i, 0)),
            pl.BlockSpec((BN // CS, BM), lambda j, i: (j, i)),
        ],
        out_shape=[
            jax.ShapeDtypeStruct((N_PAD // 128, B, 128), jnp.float32),
            jax.ShapeDtypeStruct((NCHUNK, B), jnp.float32),
        ],
    )(xa, Wa)


_RB = 512


def _tau_kernel(mt_ref, tau_ref, mrm_ref):
    m = mt_ref[...].T
    mrm_ref[...] = m
    bits = jax.lax.bitcast_convert_type(m, jnp.uint32)
    u = jnp.where(bits >= jnp.uint32(0x80000000),
                  ~bits, bits | jnp.uint32(0x80000000))
    t = jnp.zeros((_RB, 1), jnp.uint32)
    for bit in range(31, 7, -1):
        tt = t | jnp.uint32(1 << bit)
        cnt = jnp.sum((u >= tt).astype(jnp.int32), axis=1, keepdims=True)
        t = jnp.where(cnt >= TOPK, tt, t)
    fb = jnp.where(t >= jnp.uint32(0x80000000),
                   t & jnp.uint32(0x7FFFFFFF), ~t)
    tau_ref[...] = jax.lax.bitcast_convert_type(fb, jnp.float32)


def _tau(MT):
    return pl.pallas_call(
        _tau_kernel,
        grid=(B // _RB,),
        in_specs=[pl.BlockSpec((NCHUNK, _RB), lambda i: (0, i))],
        out_specs=[
            pl.BlockSpec((_RB, 1), lambda i: (i, 0)),
            pl.BlockSpec((_RB, NCHUNK), lambda i: (i, 0)),
        ],
        out_shape=[
            jax.ShapeDtypeStruct((B, 1), jnp.float32),
            jax.ShapeDtypeStruct((B, NCHUNK), jnp.float32),
        ],
    )(MT)


def _ce_desc(ka, va, kb, vb):
    ge = ka >= kb
    return (jnp.where(ge, ka, kb), jnp.where(ge, va, vb),
            jnp.where(ge, kb, ka), jnp.where(ge, vb, va))


def _sort256_desc(keys, vals):
    for i in range(16):
        keys[i], vals[i] = plsc.sort_key_val(keys[i], vals[i],
                                             descending=True)
    m = 1
    while m < 16:
        for lo in range(0, 16, 2 * m):
            seg = list(range(lo + m, lo + 2 * m))
            rk = [lax.rev(keys[i], (0,)) for i in reversed(seg)]
            rv = [lax.rev(vals[i], (0,)) for i in reversed(seg)]
            for n, i in enumerate(seg):
                keys[i], vals[i] = rk[n], rv[n]
            d = m
            while d >= 1:
                for i in range(lo, lo + 2 * m):
                    if (i - lo) % (2 * d) < d:
                        keys[i], vals[i], keys[i + d], vals[i + d] = _ce_desc(
                            keys[i], vals[i], keys[i + d], vals[i + d])
                d //= 2
            for i in range(lo, lo + 2 * m):
                keys[i], vals[i] = plsc.sort_key_val(keys[i], vals[i],
                                                     descending=True)
        m *= 2
    return keys, vals


MAXCH = 160


def _take16(v, idx):
    dn = lax.GatherDimensionNumbers(
        offset_dims=(), collapsed_slice_dims=(0,), start_index_map=(0,))
    return lax.gather(v, idx[:, None], dn, (1,),
                      mode=lax.GatherScatterMode.PROMISE_IN_BOUNDS)


def _sc_topk_body(scores_hbm, m_hbm, tau_hbm, out_hbm,
                  m_row, tau_blk, gidx_s, gidx, gidx2, colbase, gathered,
                  li_buf, out_row, sem):
    cc = lax.axis_index("c")
    ss = lax.axis_index("s")
    wid = ss * 2 + cc
    row0 = wid * ROWS_PER_W
    pltpu.sync_copy(tau_hbm.at[pl.ds(row0, ROWS_PER_W)], tau_blk)
    iota = lax.iota(jnp.int32, 16)
    zeros16 = jnp.zeros((16,), jnp.int32)
    neg_inf = jnp.full((16,), -jnp.inf, jnp.float32)

    def row_body(rr, _):
        r = row0 + rr
        pltpu.sync_copy(m_hbm.at[r], m_row)
        tau_vec = plsc.load_gather(tau_blk, [zeros16 + rr])

        pad_gidx = zeros16 + (((NCHUNK - 1) >> 2) * (4 * B) + r * 4 + 3)
        for k in range(MAXCH // 16 - 8):
            gidx_s[pl.ds(128 + k * 16, 16)] = pad_gidx
            colbase[pl.ds(128 + k * 16, 16)] = (NCHUNK - 1) * CS + zeros16

        def cid_body(ci, off):
            m16 = m_row[pl.ds(ci * 16, 16)]
            msk = jnp.logical_and(m16 >= tau_vec, (zeros16 + off) < MAXCH)
            cid = ci * 16 + iota
            gi = (cid >> 2) * (4 * B) + (cid & 3) + r * 4
            plsc.store_compressed(gidx_s.at[pl.ds(off, 16)], gi, mask=msk)
            plsc.store_compressed(colbase.at[pl.ds(off, 16)],
                                  cid * CS, mask=msk)
            return off + plsc.all_reduce_population_count(msk)[0]

        nch = lax.fori_loop(0, NCHUNK // 16, cid_body, 0)
        for k in range(8):
            gidx[pl.ds(k * 16, 16)] = gidx_s[pl.ds(k * 16, 16)]
        for k in range(MAXCH // 16 - 8):
            gidx2[pl.ds(k * 16, 16)] = gidx_s[pl.ds(128 + k * 16, 16)]

        pltpu.async_copy(scores_hbm.at[gidx],
                         gathered.at[pl.ds(0, 128)], sem).wait()

        @pl.when(nch > 128)
        def _():
            pltpu.async_copy(scores_hbm.at[gidx2],
                             gathered.at[pl.ds(128, MAXCH - 128)],
                             sem).wait()

        def scan_g(g, off):
            base = g * CS
            for k in range(CS // 16):
                sv = gathered[g, pl.ds(k * 16, 16)]
                msk = jnp.logical_and(sv >= tau_vec, (zeros16 + off) < CAP)
                liv = base + (k * 16) + iota
                plsc.store_compressed(li_buf.at[pl.ds(off, 16)], liv,
                                      mask=msk)
                off = off + plsc.all_reduce_population_count(msk)[0]
            return off

        noff = lax.fori_loop(0, jnp.minimum(nch, MAXCH), scan_g, 0)

        keys, vals = [], []
        for i in range(16):
            pos = i * 16 + iota
            valid = pos < (zeros16 + noff)
            li = jnp.where(valid, li_buf[pl.ds(i * 16, 16)], 0)
            g = li >> 5
            col = li & 31
            sc = plsc.load_gather(gathered, [g, col])
            cb = plsc.load_gather(colbase, [g])
            keys.append(jnp.where(valid, sc, neg_inf))
            vals.append(jnp.where(valid, cb + col, zeros16))
        keys, vals = _sort256_desc(keys, vals)

        shf = (iota + 1) & 15
        shb = (iota - 1) & 15
        is15 = iota == 15
        is0 = iota == 0
        pos_inf = jnp.full((16,), jnp.inf, jnp.float32)
        for i in range(8):
            kc, vc = keys[i], vals[i]
            kn = jnp.where(is15, _take16(keys[i + 1], zeros16),
                           _take16(kc, shf))
            vn = jnp.where(is15, _take16(vals[i + 1], zeros16),
                           _take16(vc, shf))
            if i == 0:
                kp, vp = pos_inf, zeros16
            else:
                kp = jnp.where(is0, _take16(keys[i - 1], zeros16 + 15),
                               _take16(kc, shb))
                vp = jnp.where(is0, _take16(vals[i - 1], zeros16 + 15),
                               _take16(vc, shb))
            fwd = jnp.logical_and(kc == kn, vc > vn)
            bwd = jnp.logical_and(kc == kp, vp > vc)
            out_row[pl.ds(i * 16, 16)] = jnp.where(
                bwd, vp, jnp.where(fwd, vn, vc))
        pltpu.sync_copy(out_row, out_hbm.at[r])
        return 0

    lax.fori_loop(0, ROWS_PER_W, row_body, 0)


def _sc_topk(scores2d, M, tau):
    mesh = plsc.VectorSubcoreMesh(core_axis_name="c", subcore_axis_name="s")
    f = functools.partial(
        pl.kernel,
        mesh=mesh,
        compiler_params=pltpu.CompilerParams(needs_layout_passes=False,
                                             use_tc_tiling_on_sc=False),
        out_type=jax.ShapeDtypeStruct((B, TOPK), jnp.int32),
        scratch_types=[
            pltpu.VMEM((NCHUNK,), jnp.float32),
            pltpu.VMEM((ROWS_PER_W,), jnp.float32),
            pltpu.VMEM((MAXCH + 16,), jnp.int32),
            pltpu.VMEM((128,), jnp.int32),
            pltpu.VMEM((MAXCH - 128,), jnp.int32),
            pltpu.VMEM((MAXCH + 16,), jnp.int32),
            pltpu.VMEM((MAXCH, CS), jnp.float32),
            pltpu.VMEM((CAP + 16,), jnp.int32),
            pltpu.VMEM((TOPK,), jnp.int32),
            pltpu.SemaphoreType.DMA,
        ],
    )(_sc_topk_body)
    return f(scores2d, M, tau)


def kernel(x, W, b):
    ones = jnp.ones((B, 1), dtype=x.dtype)
    xa = jnp.concatenate(
        [x, ones, jnp.zeros((B, K_PAD - D_IN - 1), x.dtype)], axis=-1)
    Wa = jnp.concatenate(
        [W, b[:, None], jnp.zeros((OUT_DIM, K_PAD - D_IN - 1), W.dtype)],
        axis=-1)
    Wa = jnp.pad(Wa, ((0, N_PAD - OUT_DIM), (0, 0)))
    neg = jnp.full((N_PAD - OUT_DIM,), -jnp.inf, W.dtype)
    Wa = Wa.at[OUT_DIM:, D_IN].set(neg)
    s, MT = _scores_and_maxima(xa, Wa)
    tau, Mrm = _tau(MT)
    I = _sc_topk(s.reshape(NCHUNK * B, CS), Mrm, tau.reshape(B))
    return I.astype(jnp.int64)


# speedup vs baseline: 1.4515x; 1.4515x over previous
import functools

import jax
import jax.numpy as jnp
from jax import lax
from jax.experimental import pallas as pl
from jax.experimental.pallas import tpu as pltpu
from jax.experimental.pallas import tpu_sc as plsc

B = 4096
D_IN = 128
OUT_DIM = 100000
TOPK = 128
N_PAD = 100352
K_PAD = 256
BM = 256
BN = 512
CS = 32
NCHUNK = N_PAD // CS
NW = 32
ROWS_PER_W = B // NW
CAP = 256


def _mm_kernel(x_ref, w_ref, o_ref, mt_ref):
    acc = jax.lax.dot_general(
        x_ref[...], w_ref[...], (((1,), (1,)), ((), ())),
        preferred_element_type=jnp.float32)
    for c in range(BN // 128):
        o_ref[c] = acc[:, c * 128:(c + 1) * 128]
    acc_t = jax.lax.dot_general(
        w_ref[...], x_ref[...], (((1,), (1,)), ((), ())),
        preferred_element_type=jnp.float32)
    mt_ref[...] = jnp.max(acc_t.reshape(BN // CS, CS, BM), axis=1)


def _scores_and_maxima(xa, Wa):
    grid = (N_PAD // BN, B // BM)
    return pl.pallas_call(
        _mm_kernel,
        grid=grid,
        in_specs=[
            pl.BlockSpec((BM, K_PAD), lambda j, i: (i, 0)),
            pl.BlockSpec((BN, K_PAD), lambda j, i: (j, 0)),
        ],
        out_specs=[
            pl.BlockSpec((BN // 128, BM, 128), lambda j, i: (j, i, 0)),
            pl.BlockSpec((BN // CS, BM), lambda j, i: (j, i)),
        ],
        out_shape=[
            jax.ShapeDtypeStruct((N_PAD // 128, B, 128), jnp.float32),
            jax.ShapeDtypeStruct((NCHUNK, B), jnp.float32),
        ],
    )(xa, Wa)


_RB = 512


def _tau_kernel(mt_ref, tau_ref, mrm_ref):
    m = mt_ref[...].T
    mrm_ref[...] = m
    bits = jax.lax.bitcast_convert_type(m, jnp.uint32)
    u = jnp.where(bits >= jnp.uint32(0x80000000),
                  ~bits, bits | jnp.uint32(0x80000000))
    t = jnp.zeros((_RB, 1), jnp.uint32)
    for bit in range(31, 7, -1):
        tt = t | jnp.uint32(1 << bit)
        cnt = jnp.sum((u >= tt).astype(jnp.int32), axis=1, keepdims=True)
        t = jnp.where(cnt >= TOPK, tt, t)
    t = jnp.where(t >= jnp.uint32(0x100), t - jnp.uint32(0x100),
                  jnp.uint32(0))
    fb = jnp.where(t >= jnp.uint32(0x80000000),
                   t & jnp.uint32(0x7FFFFFFF), ~t)
    tau_ref[...] = jax.lax.bitcast_convert_type(fb, jnp.float32)


def _tau(MT):
    return pl.pallas_call(
        _tau_kernel,
        grid=(B // _RB,),
        in_specs=[pl.BlockSpec((NCHUNK, _RB), lambda i: (0, i))],
        out_specs=[
            pl.BlockSpec((_RB, 1), lambda i: (i, 0)),
            pl.BlockSpec((_RB, NCHUNK), lambda i: (i, 0)),
        ],
        out_shape=[
            jax.ShapeDtypeStruct((B, 1), jnp.float32),
            jax.ShapeDtypeStruct((B, NCHUNK), jnp.float32),
        ],
    )(MT)


def _ce_desc(ka, va, kb, vb):
    ge = ka >= kb
    return (jnp.where(ge, ka, kb), jnp.where(ge, va, vb),
            jnp.where(ge, kb, ka), jnp.where(ge, vb, va))


def _sort256_desc(keys, vals):
    for i in range(16):
        keys[i], vals[i] = plsc.sort_key_val(keys[i], vals[i],
                                             descending=True)
    m = 1
    while m < 16:
        for lo in range(0, 16, 2 * m):
            seg = list(range(lo + m, lo + 2 * m))
            rk = [lax.rev(keys[i], (0,)) for i in reversed(seg)]
            rv = [lax.rev(vals[i], (0,)) for i in reversed(seg)]
            for n, i in enumerate(seg):
                keys[i], vals[i] = rk[n], rv[n]
            d = m
            while d >= 1:
                for i in range(lo, lo + 2 * m):
                    if (i - lo) % (2 * d) < d:
                        keys[i], vals[i], keys[i + d], vals[i + d] = _ce_desc(
                            keys[i], vals[i], keys[i + d], vals[i + d])
                d //= 2
            for i in range(lo, lo + 2 * m):
                keys[i], vals[i] = plsc.sort_key_val(keys[i], vals[i],
                                                     descending=True)
        m *= 2
    return keys, vals


MAXCH = 160


def _take16(v, idx):
    dn = lax.GatherDimensionNumbers(
        offset_dims=(), collapsed_slice_dims=(0,), start_index_map=(0,))
    return lax.gather(v, idx[:, None], dn, (1,),
                      mode=lax.GatherScatterMode.PROMISE_IN_BOUNDS)


def _sc_topk_body(scores_hbm, m_hbm, tau_hbm, out_hbm,
                  m_row, tau_blk, gidx_s, gidx, gidx2, colbase, gathered,
                  li_buf, out_row, sem):
    cc = lax.axis_index("c")
    ss = lax.axis_index("s")
    wid = ss * 2 + cc
    row0 = wid * ROWS_PER_W
    pltpu.sync_copy(tau_hbm.at[pl.ds(row0, ROWS_PER_W)], tau_blk)
    iota = lax.iota(jnp.int32, 16)
    zeros16 = jnp.zeros((16,), jnp.int32)
    neg_inf = jnp.full((16,), -jnp.inf, jnp.float32)

    def row_body(rr, _):
        r = row0 + rr
        pltpu.sync_copy(m_hbm.at[r], m_row)
        tau_vec = plsc.load_gather(tau_blk, [zeros16 + rr])

        pad_gidx = zeros16 + (((NCHUNK - 1) >> 2) * (4 * B) + r * 4 + 3)
        for k in range(MAXCH // 16 - 8):
            gidx_s[pl.ds(128 + k * 16, 16)] = pad_gidx
            colbase[pl.ds(128 + k * 16, 16)] = (NCHUNK - 1) * CS + zeros16

        def cid_body(ci, off):
            m16 = m_row[pl.ds(ci * 16, 16)]
            msk = jnp.logical_and(m16 >= tau_vec, (zeros16 + off) < MAXCH)
            cid = ci * 16 + iota
            gi = (cid >> 2) * (4 * B) + (cid & 3) + r * 4
            plsc.store_compressed(gidx_s.at[pl.ds(off, 16)], gi, mask=msk)
            plsc.store_compressed(colbase.at[pl.ds(off, 16)],
                                  cid * CS, mask=msk)
            return off + plsc.all_reduce_population_count(msk)[0]

        nch = lax.fori_loop(0, NCHUNK // 16, cid_body, 0)
        for k in range(8):
            gidx[pl.ds(k * 16, 16)] = gidx_s[pl.ds(k * 16, 16)]
        for k in range(MAXCH // 16 - 8):
            gidx2[pl.ds(k * 16, 16)] = gidx_s[pl.ds(128 + k * 16, 16)]

        pltpu.async_copy(scores_hbm.at[gidx],
                         gathered.at[pl.ds(0, 128)], sem).wait()

        @pl.when(nch > 128)
        def _():
            pltpu.async_copy(scores_hbm.at[gidx2],
                             gathered.at[pl.ds(128, MAXCH - 128)],
                             sem).wait()

        def scan_g(g, off):
            base = g * CS
            for k in range(CS // 16):
                sv = gathered[g, pl.ds(k * 16, 16)]
                msk = jnp.logical_and(sv >= tau_vec, (zeros16 + off) < CAP)
                liv = base + (k * 16) + iota
                plsc.store_compressed(li_buf.at[pl.ds(off, 16)], liv,
                                      mask=msk)
                off = off + plsc.all_reduce_population_count(msk)[0]
            return off

        noff = lax.fori_loop(0, jnp.minimum(nch, MAXCH), scan_g, 0)

        keys, vals = [], []
        for i in range(16):
            pos = i * 16 + iota
            valid = pos < (zeros16 + noff)
            li = jnp.where(valid, li_buf[pl.ds(i * 16, 16)], 0)
            g = li >> 5
            col = li & 31
            sc = plsc.load_gather(gathered, [g, col])
            cb = plsc.load_gather(colbase, [g])
            keys.append(jnp.where(valid, sc, neg_inf))
            vals.append(jnp.where(valid, cb + col, zeros16))
        keys, vals = _sort256_desc(keys, vals)

        shf = (iota + 1) & 15
        shb = (iota - 1) & 15
        is15 = iota == 15
        is0 = iota == 0
        pos_inf = jnp.full((16,), jnp.inf, jnp.float32)
        for i in range(8):
            kc, vc = keys[i], vals[i]
            kn = jnp.where(is15, _take16(keys[i + 1], zeros16),
                           _take16(kc, shf))
            vn = jnp.where(is15, _take16(vals[i + 1], zeros16),
                           _take16(vc, shf))
            if i == 0:
                kp, vp = pos_inf, zeros16
            else:
                kp = jnp.where(is0, _take16(keys[i - 1], zeros16 + 15),
                               _take16(kc, shb))
                vp = jnp.where(is0, _take16(vals[i - 1], zeros16 + 15),
                               _take16(vc, shb))
            fwd = jnp.logical_and(kc == kn, vc > vn)
            bwd = jnp.logical_and(kc == kp, vp > vc)
            out_row[pl.ds(i * 16, 16)] = jnp.where(
                bwd, vp, jnp.where(fwd, vn, vc))
        pltpu.sync_copy(out_row, out_hbm.at[r])
        return 0

    lax.fori_loop(0, ROWS_PER_W, row_body, 0)


def _sc_topk(scores2d, M, tau):
    mesh = plsc.VectorSubcoreMesh(core_axis_name="c", subcore_axis_name="s")
    f = functools.partial(
        pl.kernel,
        mesh=mesh,
        compiler_params=pltpu.CompilerParams(needs_layout_passes=False,
                                             use_tc_tiling_on_sc=False),
        out_type=jax.ShapeDtypeStruct((B, TOPK), jnp.int32),
        scratch_types=[
            pltpu.VMEM((NCHUNK,), jnp.float32),
            pltpu.VMEM((ROWS_PER_W,), jnp.float32),
            pltpu.VMEM((MAXCH + 16,), jnp.int32),
            pltpu.VMEM((128,), jnp.int32),
            pltpu.VMEM((MAXCH - 128,), jnp.int32),
            pltpu.VMEM((MAXCH + 16,), jnp.int32),
            pltpu.VMEM((MAXCH, CS), jnp.float32),
            pltpu.VMEM((CAP + 16,), jnp.int32),
            pltpu.VMEM((TOPK,), jnp.int32),
            pltpu.SemaphoreType.DMA,
        ],
    )(_sc_topk_body)
    return f(scores2d, M, tau)


def kernel(x, W, b):
    ones = jnp.ones((B, 1), dtype=x.dtype)
    xa = jnp.concatenate(
        [x, ones, jnp.zeros((B, K_PAD - D_IN - 1), x.dtype)], axis=-1)
    Wa = jnp.concatenate(
        [W, b[:, None], jnp.zeros((OUT_DIM, K_PAD - D_IN - 1), W.dtype)],
        axis=-1)
    Wa = jnp.pad(Wa, ((0, N_PAD - OUT_DIM), (0, 0)))
    neg = jnp.full((N_PAD - OUT_DIM,), -jnp.inf, W.dtype)
    Wa = Wa.at[OUT_DIM:, D_IN].set(neg)
    s, MT = _scores_and_maxima(xa, Wa)
    tau, Mrm = _tau(MT)
    I = _sc_topk(s.reshape(NCHUNK * B, CS), Mrm, tau.reshape(B))
    return I.astype(jnp.int64)
